# Initial kernel scaffold; baseline (speedup 1.0000x reference)
import functools

import jax
import jax.numpy as jnp
from jax import lax
from jax.experimental import pallas as pl
from jax.experimental.pallas import tpu as pltpu
from jax.experimental.pallas import tpu_sc as plsc

N = 10000
E = 320000
D = 128
DW1 = 144
NC = 2
NS = 16
CHUNK = 128
EPT = 20224
NCH = EPT // CHUNK
RPT = N // NS


def _sc_agg_body(table, srcs, dsts, zeros, out,
                 src_v, dst_v, buf0, buf1, acc, sem0, sem1):
    cid = lax.axis_index("c")
    sid = lax.axis_index("s")
    wid = sid * NC + cid

    pltpu.sync_copy(zeros, acc.at[pl.ds(sid * RPT, RPT)])
    pltpu.sync_copy(srcs.at[wid], src_v)
    pltpu.sync_copy(dsts.at[wid], dst_v)
    plsc.subcore_barrier()

    pltpu.async_copy(table.at[src_v.at[0]], buf0, sem0)

    def step(g, carry):
        j = 2 * g
        pltpu.async_copy(table.at[src_v.at[j + 1]], buf1, sem1)
        pltpu.make_async_copy(table.at[src_v.at[j]], buf0, sem0).wait()
        pltpu.sync_copy(buf0, acc.at[dst_v.at[j]], add=True)

        @pl.when(j + 2 < NCH)
        def _():
            pltpu.async_copy(table.at[src_v.at[j + 2]], buf0, sem0)

        pltpu.make_async_copy(table.at[src_v.at[j + 1]], buf1, sem1).wait()
        pltpu.sync_copy(buf1, acc.at[dst_v.at[j + 1]], add=True)
        return carry

    lax.fori_loop(0, NCH // 2, step, 0)

    plsc.subcore_barrier()
    pltpu.sync_copy(acc.at[pl.ds(sid * RPT, RPT)],
                    out.at[cid, pl.ds(sid * RPT, RPT)])


def _make_sc_agg(dw):
    mesh = plsc.VectorSubcoreMesh(core_axis_name="c", subcore_axis_name="s")
    return pl.kernel(
        _sc_agg_body,
        out_type=jax.ShapeDtypeStruct((NC, N, dw), jnp.float32),
        mesh=mesh,
        scratch_types=[
            pltpu.VMEM((NCH, CHUNK), jnp.int32),
            pltpu.VMEM((NCH, CHUNK), jnp.int32),
            pltpu.VMEM((CHUNK, dw), jnp.float32),
            pltpu.VMEM((CHUNK, dw), jnp.float32),
            pltpu.VMEM_SHARED((N, dw), jnp.float32),
            pltpu.SemaphoreType.DMA,
            pltpu.SemaphoreType.DMA,
        ],
    )


def _tc_layer1_body(agg_ref, x_ref, wl_ref, wr_ref, b_ref, h_ref, inv_ref):
    p = agg_ref[0]
    agg = p[:, :D]
    deg = jnp.sum(p[:, D:DW1], axis=1, keepdims=True)
    inv = 1.0 / jnp.maximum(deg, 1.0)
    z = jnp.dot(agg * inv, wl_ref[0], preferred_element_type=jnp.float32)
    z = z + jnp.dot(x_ref[0], wr_ref[0], preferred_element_type=jnp.float32)
    h_ref[0] = jnp.tanh(z + b_ref[0])
    inv_ref[0] = inv


def _tc_layer2_body(agg_ref, h_ref, inv_ref, wl_ref, wr_ref, b_ref, out_ref):
    z = jnp.dot(agg_ref[0] * inv_ref[0], wl_ref[0],
                preferred_element_type=jnp.float32)
    z = z + jnp.dot(h_ref[0], wr_ref[0], preferred_element_type=jnp.float32)
    out_ref[0] = jnp.tanh(z + b_ref[0])


BLK = 1000


def _make_tc_layer1():
    nblk = N // BLK
    return pl.pallas_call(
        _tc_layer1_body,
        grid=(NC, nblk),
        in_specs=[
            pl.BlockSpec((1, BLK, DW1), lambda g, i: (g, i, 0)),
            pl.BlockSpec((1, BLK, D), lambda g, i: (g, i, 0)),
            pl.BlockSpec((1, D, D), lambda g, i: (g, 0, 0)),
            pl.BlockSpec((1, D, D), lambda g, i: (g, 0, 0)),
            pl.BlockSpec((1, 1, D), lambda g, i: (g, 0, 0)),
        ],
        out_specs=[
            pl.BlockSpec((1, BLK, D), lambda g, i: (g, i, 0)),
            pl.BlockSpec((1, BLK, 1), lambda g, i: (g, i, 0)),
        ],
        out_shape=[
            jax.ShapeDtypeStruct((NC, N, D), jnp.float32),
            jax.ShapeDtypeStruct((NC, N, 1), jnp.float32),
        ],
    )


def _make_tc_layer2():
    nblk = N // BLK
    return pl.pallas_call(
        _tc_layer2_body,
        grid=(NC, nblk),
        in_specs=[
            pl.BlockSpec((1, BLK, D), lambda g, i: (g, i, 0)),
            pl.BlockSpec((1, BLK, D), lambda g, i: (g, i, 0)),
            pl.BlockSpec((1, BLK, 1), lambda g, i: (g, i, 0)),
            pl.BlockSpec((1, D, D), lambda g, i: (g, 0, 0)),
            pl.BlockSpec((1, D, D), lambda g, i: (g, 0, 0)),
            pl.BlockSpec((1, 1, D), lambda g, i: (g, 0, 0)),
        ],
        out_specs=pl.BlockSpec((1, BLK, D), lambda g, i: (g, i, 0)),
        out_shape=jax.ShapeDtypeStruct((NC, N, D), jnp.float32),
    )


def _prep_edges(edge_index0, edge_index1):
    epad = NS * EPT - E
    outs = []
    for g, ei in ((0, edge_index0), (1, edge_index1)):
        src = jnp.concatenate([ei[0], jnp.full((epad,), N, jnp.int32)])
        dst = jnp.concatenate([ei[1], jnp.zeros((epad,), jnp.int32)])
        outs.append((src.reshape(NS, NCH, CHUNK) + g * (N + 1),
                     dst.reshape(NS, NCH, CHUNK)))
    srcs = jnp.stack([outs[0][0], outs[1][0]], axis=1).reshape(NC * NS, NCH, CHUNK)
    dsts = jnp.stack([outs[0][1], outs[1][1]], axis=1).reshape(NC * NS, NCH, CHUNK)
    return srcs, dsts


def kernel(x0, edge_index0, x1, edge_index1,
           g0_W0l, g0_W0r, g0_b0, g0_W1l, g0_W1r, g0_b1,
           g1_W0l, g1_W0r, g1_b0, g1_W1l, g1_W1r, g1_b1):
    srcs, dsts = _prep_edges(edge_index0, edge_index1)

    def aug(x):
        one = jnp.ones((N, 1), jnp.float32)
        zpad = jnp.zeros((N, DW1 - D - 1), jnp.float32)
        t = jnp.concatenate([x, one, zpad], axis=1)
        return jnp.concatenate([t, jnp.zeros((1, DW1), jnp.float32)], axis=0)

    table1 = jnp.concatenate([aug(x0), aug(x1)], axis=0)
    zeros1 = jnp.zeros((RPT, DW1), jnp.float32)
    agg1 = _make_sc_agg(DW1)(table1, srcs, dsts, zeros1)

    xs = jnp.stack([x0, x1])
    wl1 = jnp.stack([g0_W0l, g1_W0l])
    wr1 = jnp.stack([g0_W0r, g1_W0r])
    bb1 = jnp.stack([g0_b0, g1_b0]).reshape(NC, 1, D)
    h, inv = _make_tc_layer1()(agg1, xs, wl1, wr1, bb1)

    zrow = jnp.zeros((NC, 1, D), jnp.float32)
    table2 = jnp.concatenate([h, zrow], axis=1).reshape(NC * (N + 1), D)
    zeros2 = jnp.zeros((RPT, D), jnp.float32)
    agg2 = _make_sc_agg(D)(table2, srcs, dsts, zeros2)

    wl2 = jnp.stack([g0_W1l, g1_W1l])
    wr2 = jnp.stack([g0_W1r, g1_W1r])
    bb2 = jnp.stack([g0_b1, g1_b1]).reshape(NC, 1, D)
    out = _make_tc_layer2()(agg2, h, inv, wl2, wr2, bb2)
    return out.reshape(NC * N, D)


# speedup vs baseline: 2.9689x; 2.9689x over previous
import functools

import jax
import jax.numpy as jnp
from jax import lax
from jax.experimental import pallas as pl
from jax.experimental.pallas import tpu as pltpu
from jax.experimental.pallas import tpu_sc as plsc

N = 10000
E = 320000
D = 128
DWA = 80
DWB = 64
NC = 2
NS = 16
EB = 128
EPT = 20480
NB = EPT // EB
NBUF_A = 3
NBUF_B = 4
RPT = 632
RPT_LAST = N - 15 * RPT


def _sc_agg_body(nbuf, table, srcs, dsts, zeros, out,
                 src_v, dst_v, bufs, acc, sg, ss):
    ga = nbuf - 2
    cid = lax.axis_index("c")
    sid = lax.axis_index("s")
    wid = sid * NC + cid

    @pl.when(sid < NS - 1)
    def _():
        pltpu.sync_copy(zeros, acc.at[pl.ds(sid * RPT, RPT)])

    @pl.when(sid == NS - 1)
    def _():
        pltpu.sync_copy(zeros.at[pl.ds(0, RPT_LAST)],
                        acc.at[pl.ds(sid * RPT, RPT_LAST)])

    pltpu.sync_copy(srcs.at[wid], src_v)
    pltpu.sync_copy(dsts.at[wid], dst_v)
    plsc.subcore_barrier()

    do = nbuf - ga
    for j0 in range(ga):
        pltpu.async_copy(table.at[src_v.at[j0]], bufs.at[j0], sg)

    def substep(j, p, jstatic):
        nxt = (p + ga) % nbuf

        def drain():
            pltpu.make_async_copy(
                bufs.at[nxt], acc.at[dst_v.at[j - do]], ss).wait()

        def prefetch():
            pltpu.async_copy(table.at[src_v.at[j + ga]], bufs.at[nxt], sg)

        if jstatic:
            if j >= do:
                drain()
            if j + ga < NB:
                prefetch()
        else:
            pl.when(j >= do)(drain)
            pl.when(j + ga < NB)(prefetch)

        pltpu.make_async_copy(table.at[src_v.at[j]], bufs.at[p], sg).wait()
        pltpu.async_copy(bufs.at[p], acc.at[dst_v.at[j]], ss, add=True)

    def step(k, carry):
        for p in range(nbuf):
            substep(nbuf * k + p, p, False)
        return carry

    nmain = (NB // nbuf) * nbuf
    lax.fori_loop(0, NB // nbuf, step, 0)
    for j in range(nmain, NB):
        substep(j, j % nbuf, True)
    for j in range(NB - do, NB):
        pltpu.make_async_copy(
            bufs.at[j % nbuf], acc.at[dst_v.at[j]], ss).wait()

    plsc.subcore_barrier()

    @pl.when(sid < NS - 1)
    def _():
        pltpu.sync_copy(acc.at[pl.ds(sid * RPT, RPT)],
                        out.at[cid, pl.ds(sid * RPT, RPT)])

    @pl.when(sid == NS - 1)
    def _():
        pltpu.sync_copy(acc.at[pl.ds(sid * RPT, RPT_LAST)],
                        out.at[cid, pl.ds(sid * RPT, RPT_LAST)])


def _make_sc_agg(dw, nbuf):
    mesh = plsc.VectorSubcoreMesh(core_axis_name="c", subcore_axis_name="s",
                                  num_cores=NC, num_subcores=NS)
    return pl.kernel(
        functools.partial(_sc_agg_body, nbuf),
        out_type=jax.ShapeDtypeStruct((NC, N, dw), jnp.float32),
        mesh=mesh,
        scratch_types=[
            pltpu.VMEM((NB, EB), jnp.int32),
            pltpu.VMEM((NB, EB), jnp.int32),
            pltpu.VMEM((nbuf, EB, dw), jnp.float32),
            pltpu.VMEM_SHARED((N, dw), jnp.float32),
            pltpu.SemaphoreType.DMA,
            pltpu.SemaphoreType.DMA,
        ],
        compiler_params=pltpu.CompilerParams(use_tc_tiling_on_sc=False, internal_scratch_in_bytes=2 * 1024 * 1024),
    )


_SC_CACHE = {}


def _sc_agg(dw, nbuf):
    if dw not in _SC_CACHE:
        _SC_CACHE[dw] = _make_sc_agg(dw, nbuf)
    return _SC_CACHE[dw]


def _tc_layer1_body(a_ref, b_ref, x_ref, wl_ref, wr_ref, b_ref_, h_ref,
                    inv_ref):
    agg = jnp.concatenate([a_ref[0], b_ref[0, :, :D - DWA]], axis=1)
    deg = jnp.sum(b_ref[0, :, D - DWA:], axis=1, keepdims=True)
    inv = 1.0 / jnp.maximum(deg, 1.0)
    z = jnp.dot(agg * inv, wl_ref[0], preferred_element_type=jnp.float32)
    z = z + jnp.dot(x_ref[0], wr_ref[0], preferred_element_type=jnp.float32)
    h_ref[0] = jnp.tanh(z + b_ref_[0])
    inv_ref[0] = inv


def _tc_layer2_body(a_ref, b_ref, h_ref, inv_ref, wl_ref, wr_ref, b_ref_,
                    out_ref):
    agg = jnp.concatenate([a_ref[0], b_ref[0, :, :D - DWA]], axis=1)
    z = jnp.dot(agg * inv_ref[0], wl_ref[0],
                preferred_element_type=jnp.float32)
    z = z + jnp.dot(h_ref[0], wr_ref[0], preferred_element_type=jnp.float32)
    out_ref[0] = jnp.tanh(z + b_ref_[0])


BLK = 1000


def _make_tc_layer1():
    nblk = N // BLK
    return pl.pallas_call(
        _tc_layer1_body,
        grid=(NC, nblk),
        in_specs=[
            pl.BlockSpec((1, BLK, DWA), lambda g, i: (g, i, 0)),
            pl.BlockSpec((1, BLK, DWB), lambda g, i: (g, i, 0)),
            pl.BlockSpec((1, BLK, D), lambda g, i: (g, i, 0)),
            pl.BlockSpec((1, D, D), lambda g, i: (g, 0, 0)),
            pl.BlockSpec((1, D, D), lambda g, i: (g, 0, 0)),
            pl.BlockSpec((1, 1, D), lambda g, i: (g, 0, 0)),
        ],
        out_specs=[
            pl.BlockSpec((1, BLK, D), lambda g, i: (g, i, 0)),
            pl.BlockSpec((1, BLK, 1), lambda g, i: (g, i, 0)),
        ],
        out_shape=[
            jax.ShapeDtypeStruct((NC, N, D), jnp.float32),
            jax.ShapeDtypeStruct((NC, N, 1), jnp.float32),
        ],
    )


def _make_tc_layer2():
    nblk = N // BLK
    return pl.pallas_call(
        _tc_layer2_body,
        grid=(NC, nblk),
        in_specs=[
            pl.BlockSpec((1, BLK, DWA), lambda g, i: (g, i, 0)),
            pl.BlockSpec((1, BLK, DWB), lambda g, i: (g, i, 0)),
            pl.BlockSpec((1, BLK, D), lambda g, i: (g, i, 0)),
            pl.BlockSpec((1, BLK, 1), lambda g, i: (g, i, 0)),
            pl.BlockSpec((1, D, D), lambda g, i: (g, 0, 0)),
            pl.BlockSpec((1, D, D), lambda g, i: (g, 0, 0)),
            pl.BlockSpec((1, 1, D), lambda g, i: (g, 0, 0)),
        ],
        out_specs=pl.BlockSpec((1, BLK, D), lambda g, i: (g, i, 0)),
        out_shape=jax.ShapeDtypeStruct((NC, N, D), jnp.float32),
    )


def _prep_edges(edge_index0, edge_index1):
    epad = NS * EPT - E
    outs = []
    for g, ei in ((0, edge_index0), (1, edge_index1)):
        src = jnp.concatenate([ei[0], jnp.full((epad,), N, jnp.int32)])
        dst = jnp.concatenate([ei[1], jnp.zeros((epad,), jnp.int32)])
        outs.append((src.reshape(NS, NB, EB) + g * (N + 1),
                     dst.reshape(NS, NB, EB)))
    srcs = jnp.stack([outs[0][0], outs[1][0]],
                     axis=1).reshape(NC * NS, NB, EB)
    dsts = jnp.stack([outs[0][1], outs[1][1]],
                     axis=1).reshape(NC * NS, NB, EB)
    return srcs, dsts


def _tables(feats):
    zrowa = jnp.zeros((NC, 1, DWA), jnp.float32)
    ta = jnp.concatenate([feats[:, :, :DWA], zrowa], axis=1)
    onespad = jnp.concatenate(
        [jnp.ones((NC, N, 1), jnp.float32),
         jnp.zeros((NC, N, DWB - (D - DWA) - 1), jnp.float32)], axis=2)
    tb = jnp.concatenate([feats[:, :, DWA:], onespad], axis=2)
    tb = jnp.concatenate([tb, jnp.zeros((NC, 1, DWB), jnp.float32)], axis=1)
    return ta.reshape(NC * (N + 1), DWA), tb.reshape(NC * (N + 1), DWB)


def kernel(x0, edge_index0, x1, edge_index1,
           g0_W0l, g0_W0r, g0_b0, g0_W1l, g0_W1r, g0_b1,
           g1_W0l, g1_W0r, g1_b0, g1_W1l, g1_W1r, g1_b1):
    srcs, dsts = _prep_edges(edge_index0, edge_index1)
    zeros_a = jnp.zeros((RPT, DWA), jnp.float32)
    zeros_b = jnp.zeros((RPT, DWB), jnp.float32)

    xs = jnp.stack([x0, x1])
    ta, tb = _tables(xs)
    agg_a = _sc_agg(DWA, NBUF_A)(ta, srcs, dsts, zeros_a)
    agg_b = _sc_agg(DWB, NBUF_B)(tb, srcs, dsts, zeros_b)

    wl1 = jnp.stack([g0_W0l, g1_W0l])
    wr1 = jnp.stack([g0_W0r, g1_W0r])
    bb1 = jnp.stack([g0_b0, g1_b0]).reshape(NC, 1, D)
    h, inv = _make_tc_layer1()(agg_a, agg_b, xs, wl1, wr1, bb1)

    ta2, tb2 = _tables(h)
    agg2_a = _sc_agg(DWA, NBUF_A)(ta2, srcs, dsts, zeros_a)
    agg2_b = _sc_agg(DWB, NBUF_B)(tb2, srcs, dsts, zeros_b)

    wl2 = jnp.stack([g0_W1l, g1_W1l])
    wr2 = jnp.stack([g0_W1r, g1_W1r])
    bb2 = jnp.stack([g0_b1, g1_b1]).reshape(NC, 1, D)
    out = _make_tc_layer2()(agg2_a, agg2_b, h, inv, wl2, wr2, bb2)
    return out.reshape(NC * N, D)


# speedup vs baseline: 3.0497x; 1.0272x over previous
import functools

import jax
import jax.numpy as jnp
from jax import lax
from jax.experimental import pallas as pl
from jax.experimental.pallas import tpu as pltpu
from jax.experimental.pallas import tpu_sc as plsc

N = 10000
E = 320000
D = 128
DWA = 80
DWB = 64
NC = 2
NS = 16
EB = 128
EPT = 20480
NB = EPT // EB
NBUF_A = 3
NBUF_B = 4
RPT = 632
RPT_LAST = N - 15 * RPT


def _sc_agg_body(nbuf, table, srcs, dsts, zeros, out,
                 src_v, dst_v, bufs, acc, sg, ss):
    ga = nbuf - 2
    cid = lax.axis_index("c")
    sid = lax.axis_index("s")
    wid = sid * NC + cid

    @pl.when(sid < NS - 1)
    def _():
        pltpu.sync_copy(zeros, acc.at[pl.ds(sid * RPT, RPT)])

    @pl.when(sid == NS - 1)
    def _():
        pltpu.sync_copy(zeros.at[pl.ds(0, RPT_LAST)],
                        acc.at[pl.ds(sid * RPT, RPT_LAST)])

    pltpu.sync_copy(srcs.at[wid], src_v)
    pltpu.sync_copy(dsts.at[wid], dst_v)
    plsc.subcore_barrier()

    do = nbuf - ga
    for j0 in range(ga):
        pltpu.async_copy(table.at[src_v.at[j0]], bufs.at[j0], sg)

    def substep(j, p, jstatic):
        nxt = (p + ga) % nbuf

        def drain():
            pltpu.make_async_copy(
                bufs.at[nxt], acc.at[dst_v.at[j - do]], ss).wait()

        def prefetch():
            pltpu.async_copy(table.at[src_v.at[j + ga]], bufs.at[nxt], sg)

        if jstatic:
            if j >= do:
                drain()
            if j + ga < NB:
                prefetch()
        else:
            pl.when(j >= do)(drain)
            pl.when(j + ga < NB)(prefetch)

        pltpu.make_async_copy(table.at[src_v.at[j]], bufs.at[p], sg).wait()
        pltpu.async_copy(bufs.at[p], acc.at[dst_v.at[j]], ss, add=True)

    def step(k, carry):
        for p in range(nbuf):
            substep(nbuf * k + p, p, False)
        return carry

    nmain = (NB // nbuf) * nbuf
    lax.fori_loop(0, NB // nbuf, step, 0)
    for j in range(nmain, NB):
        substep(j, j % nbuf, True)
    for j in range(NB - do, NB):
        pltpu.make_async_copy(
            bufs.at[j % nbuf], acc.at[dst_v.at[j]], ss).wait()

    plsc.subcore_barrier()

    @pl.when(sid < NS - 1)
    def _():
        pltpu.sync_copy(acc.at[pl.ds(sid * RPT, RPT)],
                        out.at[cid, pl.ds(sid * RPT, RPT)])

    @pl.when(sid == NS - 1)
    def _():
        pltpu.sync_copy(acc.at[pl.ds(sid * RPT, RPT_LAST)],
                        out.at[cid, pl.ds(sid * RPT, RPT_LAST)])


def _make_sc_agg(dw, nbuf):
    mesh = plsc.VectorSubcoreMesh(core_axis_name="c", subcore_axis_name="s",
                                  num_cores=NC, num_subcores=NS)
    return pl.kernel(
        functools.partial(_sc_agg_body, nbuf),
        out_type=jax.ShapeDtypeStruct((NC, N, dw), jnp.float32),
        mesh=mesh,
        scratch_types=[
            pltpu.VMEM((NB, EB), jnp.int32),
            pltpu.VMEM((NB, EB), jnp.int32),
            pltpu.VMEM((nbuf, EB, dw), jnp.float32),
            pltpu.VMEM_SHARED((N, dw), jnp.float32),
            pltpu.SemaphoreType.DMA,
            pltpu.SemaphoreType.DMA,
        ],
        compiler_params=pltpu.CompilerParams(use_tc_tiling_on_sc=False),
    )


_SC_CACHE = {}


def _sc_agg(dw, nbuf):
    if dw not in _SC_CACHE:
        _SC_CACHE[dw] = _make_sc_agg(dw, nbuf)
    return _SC_CACHE[dw]


def _tc_layer1_body(a_ref, b_ref, x_ref, wl_ref, wr_ref, b_ref_, h_ref,
                    inv_ref):
    agg = jnp.concatenate([a_ref[0], b_ref[0, :, :D - DWA]], axis=1)
    deg = jnp.sum(b_ref[0, :, D - DWA:], axis=1, keepdims=True)
    inv = 1.0 / jnp.maximum(deg, 1.0)
    z = jnp.dot(agg * inv, wl_ref[0], preferred_element_type=jnp.float32)
    z = z + jnp.dot(x_ref[0], wr_ref[0], preferred_element_type=jnp.float32)
    h_ref[0] = jnp.tanh(z + b_ref_[0])
    inv_ref[0] = inv


def _tc_layer2_body(a_ref, b_ref, h_ref, inv_ref, wl_ref, wr_ref, b_ref_,
                    out_ref):
    agg = jnp.concatenate([a_ref[0], b_ref[0, :, :D - DWA]], axis=1)
    z = jnp.dot(agg * inv_ref[0], wl_ref[0],
                preferred_element_type=jnp.float32)
    z = z + jnp.dot(h_ref[0], wr_ref[0], preferred_element_type=jnp.float32)
    out_ref[0] = jnp.tanh(z + b_ref_[0])


BLK = 1000


def _make_tc_layer1():
    nblk = N // BLK
    return pl.pallas_call(
        _tc_layer1_body,
        grid=(NC, nblk),
        in_specs=[
            pl.BlockSpec((1, BLK, DWA), lambda g, i: (g, i, 0)),
            pl.BlockSpec((1, BLK, DWB), lambda g, i: (g, i, 0)),
            pl.BlockSpec((1, BLK, D), lambda g, i: (g, i, 0)),
            pl.BlockSpec((1, D, D), lambda g, i: (g, 0, 0)),
            pl.BlockSpec((1, D, D), lambda g, i: (g, 0, 0)),
            pl.BlockSpec((1, 1, D), lambda g, i: (g, 0, 0)),
        ],
        out_specs=[
            pl.BlockSpec((1, BLK, D), lambda g, i: (g, i, 0)),
            pl.BlockSpec((1, BLK, 1), lambda g, i: (g, i, 0)),
        ],
        out_shape=[
            jax.ShapeDtypeStruct((NC, N, D), jnp.float32),
            jax.ShapeDtypeStruct((NC, N, 1), jnp.float32),
        ],
    )


def _make_tc_layer2():
    nblk = N // BLK
    return pl.pallas_call(
        _tc_layer2_body,
        grid=(NC, nblk),
        in_specs=[
            pl.BlockSpec((1, BLK, DWA), lambda g, i: (g, i, 0)),
            pl.BlockSpec((1, BLK, DWB), lambda g, i: (g, i, 0)),
            pl.BlockSpec((1, BLK, D), lambda g, i: (g, i, 0)),
            pl.BlockSpec((1, BLK, 1), lambda g, i: (g, i, 0)),
            pl.BlockSpec((1, D, D), lambda g, i: (g, 0, 0)),
            pl.BlockSpec((1, D, D), lambda g, i: (g, 0, 0)),
            pl.BlockSpec((1, 1, D), lambda g, i: (g, 0, 0)),
        ],
        out_specs=pl.BlockSpec((1, BLK, D), lambda g, i: (g, i, 0)),
        out_shape=jax.ShapeDtypeStruct((NC, N, D), jnp.float32),
    )


def _prep_edges(edge_index0, edge_index1):
    epad = NS * EPT - E
    outs = []
    for g, ei in ((0, edge_index0), (1, edge_index1)):
        src = jnp.concatenate([ei[0], jnp.full((epad,), N, jnp.int32)])
        dst = jnp.concatenate([ei[1], jnp.zeros((epad,), jnp.int32)])
        outs.append((src.reshape(NS, NB, EB) + g * (N + 1),
                     dst.reshape(NS, NB, EB)))
    srcs = jnp.stack([outs[0][0], outs[1][0]],
                     axis=1).reshape(NC * NS, NB, EB)
    dsts = jnp.stack([outs[0][1], outs[1][1]],
                     axis=1).reshape(NC * NS, NB, EB)
    return srcs, dsts


def _tables(feats):
    zrowa = jnp.zeros((NC, 1, DWA), jnp.float32)
    ta = jnp.concatenate([feats[:, :, :DWA], zrowa], axis=1)
    onespad = jnp.concatenate(
        [jnp.ones((NC, N, 1), jnp.float32),
         jnp.zeros((NC, N, DWB - (D - DWA) - 1), jnp.float32)], axis=2)
    tb = jnp.concatenate([feats[:, :, DWA:], onespad], axis=2)
    tb = jnp.concatenate([tb, jnp.zeros((NC, 1, DWB), jnp.float32)], axis=1)
    return ta.reshape(NC * (N + 1), DWA), tb.reshape(NC * (N + 1), DWB)


def kernel(x0, edge_index0, x1, edge_index1,
           g0_W0l, g0_W0r, g0_b0, g0_W1l, g0_W1r, g0_b1,
           g1_W0l, g1_W0r, g1_b0, g1_W1l, g1_W1r, g1_b1):
    srcs, dsts = _prep_edges(edge_index0, edge_index1)
    zeros_a = jnp.zeros((RPT, DWA), jnp.float32)
    zeros_b = jnp.zeros((RPT, DWB), jnp.float32)

    xs = jnp.stack([x0, x1])
    ta, tb = _tables(xs)
    agg_a = _sc_agg(DWA, NBUF_A)(ta, srcs, dsts, zeros_a)
    agg_b = _sc_agg(DWB, NBUF_B)(tb, srcs, dsts, zeros_b)

    wl1 = jnp.stack([g0_W0l, g1_W0l])
    wr1 = jnp.stack([g0_W0r, g1_W0r])
    bb1 = jnp.stack([g0_b0, g1_b0]).reshape(NC, 1, D)
    h, inv = _make_tc_layer1()(agg_a, agg_b, xs, wl1, wr1, bb1)

    ta2, tb2 = _tables(h)
    agg2_a = _sc_agg(DWA, NBUF_A)(ta2, srcs, dsts, zeros_a)
    agg2_b = _sc_agg(DWB, NBUF_B)(tb2, srcs, dsts, zeros_b)

    wl2 = jnp.stack([g0_W1l, g1_W1l])
    wr2 = jnp.stack([g0_W1r, g1_W1r])
    bb2 = jnp.stack([g0_b1, g1_b1]).reshape(NC, 1, D)
    out = _make_tc_layer2()(agg2_a, agg2_b, h, inv, wl2, wr2, bb2)
    return out.reshape(NC * N, D)
